# Initial kernel scaffold; baseline (speedup 1.0000x reference)
import jax
import jax.numpy as jnp
from jax.experimental import pallas as pl


def kernel(x, edge_index_intra, edge_index_inter, batch, W1, b1, g1, be1, W2, b2, g2, be2, W3, b3, g3, be3, W4, b4, g4, be4, W5, b5, g5, be5, fc1W, fc1b, fc2W, fc2b):
    raise NotImplementedError("write your pallas kernel here")


# speedup vs baseline: 7.6060x; 7.6060x over previous
import functools

import jax
import jax.numpy as jnp
from jax import lax
from jax.experimental import pallas as pl
from jax.experimental.pallas import tpu as pltpu
from jax.experimental.pallas import tpu_sc as plsc

N = 10000
G = 64
NC = 2
NS = 16
NW = NC * NS
CH = 128
NCHUNK = 80
EPAD = NW * NCHUNK * CH
STRIPE = 632
NPAD = NS * STRIPE


def _deg_kernel():
    mesh = plsc.VectorSubcoreMesh(core_axis_name="c", subcore_axis_name="s")

    @functools.partial(
        pl.kernel,
        out_type=jax.ShapeDtypeStruct((NC, NPAD, 16), jnp.float32),
        mesh=mesh,
        scratch_types=[
            pltpu.VMEM((NCHUNK, CH), jnp.int32),
            pltpu.VMEM((CH, 16), jnp.float32),
            pltpu.VMEM((CH, 16), jnp.float32),
            pltpu.VMEM_SHARED((NPAD, 16), jnp.float32),
        ],
        compiler_params=pltpu.CompilerParams(use_tc_tiling_on_sc=False),
    )
    def k(dst_hbm, out_hbm, dst_v, ones_b, zero_b, acc):
        c = lax.axis_index("c")
        s = lax.axis_index("s")
        w = c * NS + s
        pltpu.sync_copy(dst_hbm.at[w], dst_v)

        def fill(i, _):
            ones_b[i, pl.ds(0, 16)] = jnp.ones((16,), jnp.float32)
            zero_b[i, pl.ds(0, 16)] = jnp.zeros((16,), jnp.float32)
            return 0

        lax.fori_loop(0, CH, fill, 0)

        r0 = s * STRIPE
        for t in range(STRIPE // CH):
            pltpu.sync_copy(zero_b, acc.at[pl.ds(r0 + t * CH, CH)])
        rem = STRIPE % CH
        if rem:
            pltpu.sync_copy(zero_b.at[pl.ds(0, rem)],
                            acc.at[pl.ds(r0 + (STRIPE // CH) * CH, rem)])
        plsc.subcore_barrier()

        def body(ch, _):
            pltpu.sync_copy(ones_b, acc.at[dst_v.at[ch]], add=True)
            return 0

        lax.fori_loop(0, NCHUNK, body, 0)
        plsc.subcore_barrier()
        pltpu.sync_copy(acc.at[pl.ds(r0, STRIPE)],
                        out_hbm.at[c, pl.ds(r0, STRIPE)])

    return k


def _make_spmm_kernel(dout):
    mesh = plsc.VectorSubcoreMesh(core_axis_name="c", subcore_axis_name="s")

    @functools.partial(
        pl.kernel,
        out_type=jax.ShapeDtypeStruct((NC, NPAD, dout), jnp.float32),
        mesh=mesh,
        scratch_types=[
            pltpu.VMEM((NCHUNK, CH), jnp.int32),
            pltpu.VMEM((NCHUNK, CH), jnp.int32),
            pltpu.VMEM((CH, dout), jnp.float32),
            pltpu.VMEM((CH, dout), jnp.float32),
            pltpu.VMEM((CH, dout), jnp.float32),
            pltpu.VMEM_SHARED((NPAD, dout), jnp.float32),
            pltpu.SemaphoreType.DMA,
            pltpu.SemaphoreType.DMA,
        ],
        compiler_params=pltpu.CompilerParams(use_tc_tiling_on_sc=False),
    )
    def k(h_hbm, src_hbm, dst_hbm, out_hbm,
          src_v, dst_v, gb0, gb1, zb, acc, sem0, sem1):
        c = lax.axis_index("c")
        s = lax.axis_index("s")
        w = c * NS + s
        pltpu.sync_copy(src_hbm.at[w], src_v)
        pltpu.sync_copy(dst_hbm.at[w], dst_v)

        def zrow(i, _):
            for j in range(dout // 16):
                zb[i, pl.ds(j * 16, 16)] = jnp.zeros((16,), jnp.float32)
            return 0

        lax.fori_loop(0, CH, zrow, 0)

        r0 = s * STRIPE
        for t in range(STRIPE // CH):
            pltpu.sync_copy(zb, acc.at[pl.ds(r0 + t * CH, CH)])
        rem = STRIPE % CH
        if rem:
            pltpu.sync_copy(zb.at[pl.ds(0, rem)],
                            acc.at[pl.ds(r0 + (STRIPE // CH) * CH, rem)])
        plsc.subcore_barrier()

        gbufs = (gb0, gb1)
        sems = (sem0, sem1)
        for b in range(2):
            pltpu.async_copy(h_hbm.at[src_v.at[b]], gbufs[b], sems[b])

        def body(i, _):
            g = i * 2
            for b in range(2):
                ch = g + b
                pltpu.make_async_copy(h_hbm.at[src_v.at[ch]], gbufs[b],
                                      sems[b]).wait()
                pltpu.sync_copy(gbufs[b], acc.at[dst_v.at[ch]], add=True)

                @pl.when(ch + 2 < NCHUNK)
                def _():
                    pltpu.async_copy(h_hbm.at[src_v.at[ch + 2]], gbufs[b],
                                     sems[b])
            return 0

        lax.fori_loop(0, NCHUNK // 2, body, 0)
        plsc.subcore_barrier()
        pltpu.sync_copy(acc.at[pl.ds(r0, STRIPE)],
                        out_hbm.at[c, pl.ds(r0, STRIPE)])

    return k


_SPMM = {}


def _spmm(hq, src3, dst3):
    dout = hq.shape[1]
    if dout not in _SPMM:
        _SPMM[dout] = _make_spmm_kernel(dout)
    return _SPMM[dout](hq, src3, dst3)


_PREC = None
_PREC_POOL = lax.Precision.HIGHEST
_RB = 2000
_NB = N // _RB


def _split_cols(W):
    d = W.shape[1]
    if d <= 64:
        return [W]
    assert d % 64 == 0
    return [W[:, i * 64:(i + 1) * 64] for i in range(d // 64)]


def _stage0_body(degp_ref, x_ref, w_ref, hq_ref, dinv_ref):
    deg = degp_ref[0, :, 0:1] + degp_ref[1, :, 0:1] + 1.0
    dinv = lax.rsqrt(deg)
    dinv_ref[...] = dinv
    h = jnp.dot(x_ref[...], w_ref[...],
                preferred_element_type=jnp.float32, precision=_PREC)
    hq_ref[...] = h * dinv


def _stage0(degp, x, W1):
    dout = W1.shape[1]
    return pl.pallas_call(
        _stage0_body,
        grid=(_NB,),
        in_specs=[
            pl.BlockSpec((2, _RB, 16), lambda i: (0, i, 0)),
            pl.BlockSpec((_RB, x.shape[1]), lambda i: (i, 0)),
            pl.BlockSpec(W1.shape, lambda i: (0, 0)),
        ],
        out_specs=[
            pl.BlockSpec((_RB, dout), lambda i: (i, 0)),
            pl.BlockSpec((_RB, 1), lambda i: (i, 0)),
        ],
        out_shape=[jax.ShapeDtypeStruct((N, dout), jnp.float32),
                   jax.ShapeDtypeStruct((N, 1), jnp.float32)],
    )(degp, x, W1)


def _conv_u(p_refs, hq_refs, dinv, b_ref, relu_before):
    cols = [p[0] + p[1] + hq[...] for p, hq in zip(p_refs, hq_refs)]
    u = cols[0] if len(cols) == 1 else jnp.concatenate(cols, axis=1)
    u = dinv * u + b_ref[...]
    if relu_before:
        u = jnp.maximum(u, 0.0)
    return u


def _layer_body(nchunk, nout, bn_first, *refs):
    p_refs = refs[:nchunk]
    hq_refs = refs[nchunk:2 * nchunk]
    dinv_ref, b_ref, g_ref, be_ref, w_ref = refs[2 * nchunk:2 * nchunk + 5]
    o_refs = refs[2 * nchunk + 5:2 * nchunk + 5 + nout]
    ssum, ssq = refs[2 * nchunk + 5 + nout:]
    ph = pl.program_id(0)
    i = pl.program_id(1)
    dinv = dinv_ref[...]
    u = _conv_u(p_refs, hq_refs, dinv, b_ref, relu_before=not bn_first)

    @pl.when(ph == 0)
    def _():
        @pl.when(i == 0)
        def _():
            ssum[...] = jnp.zeros_like(ssum)
        ssum[...] += jnp.sum(u, axis=0, keepdims=True)

    @pl.when(ph == 1)
    def _():
        m = ssum[...] * (1.0 / N)
        ctr = u - m

        @pl.when(i == 0)
        def _():
            ssq[...] = jnp.zeros_like(ssq)
        ssq[...] += jnp.sum(ctr * ctr, axis=0, keepdims=True)

    @pl.when(ph == 2)
    def _():
        m = ssum[...] * (1.0 / N)
        v = ssq[...] * (1.0 / N)
        t = (u - m) * lax.rsqrt(v + 1e-5) * g_ref[...] + be_ref[...]
        if bn_first:
            t = jnp.maximum(t, 0.0)
        res = dinv * jnp.dot(t, w_ref[...],
                             preferred_element_type=jnp.float32,
                             precision=_PREC)
        ow = res.shape[1] // nout
        for j, o_ref in enumerate(o_refs):
            o_ref[...] = res[:, j * ow:(j + 1) * ow]


def _layer(ps, hqs, dinv, b, g, be, Wn, bn_first):
    din = Wn.shape[0]
    nchunk = len(ps)
    nout = max(1, Wn.shape[1] // 64)
    outw = Wn.shape[1] // nout
    body = functools.partial(_layer_body, nchunk, nout, bn_first)
    in_specs = (
        [pl.BlockSpec((2, _RB, p.shape[2]), lambda ph, i: (0, i, 0))
         for p in ps]
        + [pl.BlockSpec((_RB, hq.shape[1]), lambda ph, i: (i, 0))
           for hq in hqs]
        + [pl.BlockSpec((_RB, 1), lambda ph, i: (i, 0))]
        + [pl.BlockSpec((1, din), lambda ph, i: (0, 0))] * 3
        + [pl.BlockSpec(Wn.shape, lambda ph, i: (0, 0))]
    )
    return pl.pallas_call(
        body,
        grid=(3, _NB),
        in_specs=in_specs,
        out_specs=[pl.BlockSpec((_RB, outw), lambda ph, i: (i, 0))
                   for _ in range(nout)],
        out_shape=[jax.ShapeDtypeStruct((N, outw), jnp.float32)
                   for _ in range(nout)],
        scratch_shapes=[pltpu.VMEM((1, din), jnp.float32),
                        pltpu.VMEM((1, din), jnp.float32)],
    )(*ps, *hqs, dinv, b.reshape(1, -1), g.reshape(1, -1), be.reshape(1, -1),
      Wn)


def _final_body(nchunk, din, *refs):
    p_refs = refs[:nchunk]
    hq_refs = refs[nchunk:2 * nchunk]
    (dinv_ref, b_ref, g_ref, be_ref, batch_ref, fc1w_ref, fc1b_ref,
     fc2w_ref, fc2b_ref, o_ref, ssum, ssq, pooled) = refs[2 * nchunk:]
    ph = pl.program_id(0)
    i = pl.program_id(1)
    dinv = dinv_ref[...]
    u = _conv_u(p_refs, hq_refs, dinv, b_ref, relu_before=False)

    @pl.when(ph == 0)
    def _():
        @pl.when(i == 0)
        def _():
            ssum[...] = jnp.zeros_like(ssum)
        ssum[...] += jnp.sum(u, axis=0, keepdims=True)

    @pl.when(ph == 1)
    def _():
        m = ssum[...] * (1.0 / N)
        ctr = u - m

        @pl.when(i == 0)
        def _():
            ssq[...] = jnp.zeros_like(ssq)
        ssq[...] += jnp.sum(ctr * ctr, axis=0, keepdims=True)

    @pl.when(ph == 2)
    def _():
        m = ssum[...] * (1.0 / N)
        v = ssq[...] * (1.0 / N)
        t = (u - m) * lax.rsqrt(v + 1e-5) * g_ref[...] + be_ref[...]
        oh = (batch_ref[...] ==
              lax.broadcasted_iota(jnp.int32, (1, G), 1)).astype(jnp.float32)
        part = lax.dot_general(oh, t, (((0,), (0,)), ((), ())),
                               preferred_element_type=jnp.float32,
                               precision=_PREC_POOL)

        @pl.when(i == 0)
        def _():
            pooled[...] = part

        @pl.when(i > 0)
        def _():
            pooled[...] += part

        @pl.when(i == _NB - 1)
        def _():
            r = jnp.maximum(pooled[...], 0.0)
            r = jnp.maximum(jnp.dot(r, fc1w_ref[...],
                                    preferred_element_type=jnp.float32,
                                    precision=_PREC) + fc1b_ref[...], 0.0)
            o_ref[...] = jnp.dot(r, fc2w_ref[...],
                                 preferred_element_type=jnp.float32,
                                 precision=_PREC) + fc2b_ref[...]


def _final(ps, hqs, dinv, b, g, be, batch, fc1W, fc1b, fc2W, fc2b):
    nchunk = len(ps)
    din = sum(hq.shape[1] for hq in hqs)
    body = functools.partial(_final_body, nchunk, din)
    in_specs = (
        [pl.BlockSpec((2, _RB, p.shape[2]), lambda ph, i: (0, i, 0))
         for p in ps]
        + [pl.BlockSpec((_RB, hq.shape[1]), lambda ph, i: (i, 0))
           for hq in hqs]
        + [pl.BlockSpec((_RB, 1), lambda ph, i: (i, 0))]
        + [pl.BlockSpec((1, din), lambda ph, i: (0, 0))] * 3
        + [pl.BlockSpec((_RB, 1), lambda ph, i: (i, 0))]
        + [pl.BlockSpec(fc1W.shape, lambda ph, i: (0, 0)),
           pl.BlockSpec((1, fc1b.shape[0]), lambda ph, i: (0, 0)),
           pl.BlockSpec(fc2W.shape, lambda ph, i: (0, 0)),
           pl.BlockSpec((1, 1), lambda ph, i: (0, 0))]
    )
    return pl.pallas_call(
        body,
        grid=(3, _NB),
        in_specs=in_specs,
        out_specs=pl.BlockSpec((G, 1), lambda ph, i: (0, 0)),
        out_shape=jax.ShapeDtypeStruct((G, 1), jnp.float32),
        scratch_shapes=[pltpu.VMEM((1, din), jnp.float32),
                        pltpu.VMEM((1, din), jnp.float32),
                        pltpu.VMEM((G, din), jnp.float32)],
    )(*ps, *hqs, dinv, b.reshape(1, -1), g.reshape(1, -1), be.reshape(1, -1),
      batch.astype(jnp.int32).reshape(-1, 1),
      fc1W, fc1b.reshape(1, -1), fc2W, fc2b.reshape(1, -1))


def kernel(x, edge_index_intra, edge_index_inter, batch,
           W1, b1, g1, be1, W2, b2, g2, be2, W3, b3, g3, be3,
           W4, b4, g4, be4, W5, b5, g5, be5, fc1W, fc1b, fc2W, fc2b):
    src = jnp.concatenate([edge_index_intra[0], edge_index_inter[0]])
    dst = jnp.concatenate([edge_index_intra[1], edge_index_inter[1]])
    src = src.astype(jnp.int32)
    dst = dst.astype(jnp.int32)
    npad = EPAD - src.shape[0]
    src3 = jnp.concatenate([src, jnp.zeros((npad,), jnp.int32)])
    dst3 = jnp.concatenate([dst, jnp.full((npad,), N, jnp.int32)])
    src3 = src3.reshape(NW, NCHUNK, CH)
    dst3 = dst3.reshape(NW, NCHUNK, CH)

    degp = _deg_kernel()(dst3)
    hq1, dinv = _stage0(degp, x, W1)
    hqs = [hq1]

    layers = [(b1, g1, be1, W2, False), (b2, g2, be2, W3, False),
              (b3, g3, be3, W4, False), (b4, g4, be4, W5, True)]
    for b, g, be, Wn, bn_first in layers:
        ps = [_spmm(hq, src3, dst3) for hq in hqs]
        hqs = _layer(ps, hqs, dinv, b, g, be, Wn, bn_first)

    ps = [_spmm(hq, src3, dst3) for hq in hqs]
    out = _final(ps, hqs, dinv, b5, g5, be5, batch, fc1W, fc1b, fc2W, fc2b)
    return out.reshape(-1)
